# baseline (speedup 1.0000x reference)
import jax
import jax.numpy as jnp
from jax import lax
from jax.experimental import pallas as pl
from jax.experimental.pallas import tpu as pltpu


_COLS_PER_BLOCK = 1024
_SUB = 8


def _score_block(logits_ref, labels_ref, out_ref):
    c, bn = logits_ref.shape
    lab = labels_ref[...]
    iota8 = jax.lax.broadcasted_iota(jnp.int32, (_SUB, bn), 0)
    tlab = jnp.broadcast_to(lab, (_SUB, bn)) - iota8

    def body1(k, carry):
        m8, xl8 = carry
        v = logits_ref[pl.ds(_SUB * k, _SUB), :]
        m8 = jnp.maximum(m8, v)
        xl8 = jnp.where(tlab == _SUB * k, v, xl8)
        return m8, xl8

    m8 = jnp.full((_SUB, bn), -jnp.inf, dtype=jnp.float32)
    xl8 = jnp.zeros((_SUB, bn), dtype=jnp.float32)
    m8, xl8 = lax.fori_loop(0, c // _SUB, body1, (m8, xl8))
    mb = jnp.broadcast_to(jnp.max(m8, axis=0, keepdims=True), (_SUB, bn))
    xlb = jnp.broadcast_to(jnp.sum(xl8, axis=0, keepdims=True), (_SUB, bn))

    def body2(k, carry):
        z8, n8 = carry
        v = logits_ref[pl.ds(_SUB * k, _SUB), :]
        e = jnp.exp(v - mb)
        mask = (v > xlb) | ((v == xlb) & (tlab >= _SUB * k))
        return z8 + e, n8 + jnp.where(mask, e, 0.0)

    z8 = jnp.zeros((_SUB, bn), dtype=jnp.float32)
    n8 = jnp.zeros((_SUB, bn), dtype=jnp.float32)
    z8, n8 = lax.fori_loop(0, c // _SUB, body2, (z8, n8))
    z = jnp.sum(z8, axis=0, keepdims=True)
    num = jnp.sum(n8, axis=0, keepdims=True)
    out_ref[...] = num / z


@jax.jit
def kernel(logits, labels):
    n, c = logits.shape
    xt = logits.T
    lab2d = labels.astype(jnp.int32).reshape(1, n)
    bn = _COLS_PER_BLOCK
    out = pl.pallas_call(
        _score_block,
        grid=(n // bn,),
        in_specs=[
            pl.BlockSpec((c, bn), lambda j: (0, j)),
            pl.BlockSpec((1, bn), lambda j: (0, j)),
        ],
        out_specs=pl.BlockSpec((1, bn), lambda j: (0, j)),
        out_shape=jax.ShapeDtypeStruct((1, n), jnp.float32),
        compiler_params=pltpu.CompilerParams(
            dimension_semantics=("parallel",),
        ),
    )(xt, lab2d)
    return out.reshape(n)
